# baseline (speedup 1.0000x reference)
import functools
import math

import jax
import jax.numpy as jnp
from jax import lax
from jax.experimental import pallas as pl
from jax.experimental.pallas import tpu as pltpu
from jax.experimental.pallas import tpu_sc as plsc

_G0, _G1, _G2, _G3, _F = 64, 64, 24, 24, 32
_N = 262144
_V = _G0 * _G1 * _G2 * _G3
_M1 = _V // 4

_NC = 2
_NS = 16
_NW = _NC * _NS
_Q = 64
_NW_Q = _N // _NW
_NB = _NW_Q // _Q

_PREP_B = 2048
_MLP_B = 16384


def _prep_body(x_ref, idx_ref, w_ref):
    pi = math.pi
    x0 = x_ref[0]
    x1 = x_ref[1]
    x2 = x_ref[2]
    x3 = x_ref[3]
    t0 = (x0 - 0.0) / (pi - 0.0) * (_G0 - 1)
    t1 = (x1 - (-pi)) / (pi - (-pi)) * (_G1 - 1)
    t2 = (x2 - 0.5 * pi) / (0.85 * pi - 0.5 * pi) * (_G2 - 1)
    t3 = (x3 - (-0.85 * pi)) / (-0.5 * pi - (-0.85 * pi)) * (_G3 - 1)
    tlx = t0.astype(jnp.int32)
    tly = t1.astype(jnp.int32)
    tlz = t2.astype(jnp.int32)
    tlw = t3.astype(jnp.int32)
    xf = t0 % 1
    yf = t1 % 1
    zf = t2 % 1
    wf = t3 % 1
    brx = jnp.minimum(tlx + 1, _G1 - 1)
    bry = jnp.minimum(tly + 1, _G0 - 1)
    brz = jnp.minimum(tlz + 1, _G2 - 1)
    brw = jnp.minimum(tlw + 1, _G3 - 1)
    ws = []
    for k in range(16):
        bx, by, bz, bw = (k >> 3) & 1, (k >> 2) & 1, (k >> 1) & 1, k & 1
        iy = bry if by else tly
        ix = brx if bx else tlx
        iz = brz if bz else tlz
        iw = brw if bw else tlw
        flat = ((iy * _G1 + ix) * _G2 + iz) * _G3 + iw
        c = ((flat >= _M1).astype(jnp.int32)
             + (flat >= 2 * _M1).astype(jnp.int32)
             + (flat >= 3 * _M1).astype(jnp.int32))
        flat = (flat - c * _M1) * 4 + c
        ax = xf if bx else 1.0 - xf
        ay = yf if by else 1.0 - yf
        zsel, wsel = (bw, bz) if by else (bz, bw)
        az = zf if zsel else 1.0 - zf
        aw = wf if wsel else 1.0 - wf
        idx_ref[k] = flat
        w_ref[k] = ax * ay * az * aw


_PREP_ROWS = 64


def _prep_call(x):
    nrows = _N // 128
    xt = x.T.reshape(4, nrows, 128)
    idx3, w3 = pl.pallas_call(
        _prep_body,
        grid=(nrows // _PREP_ROWS,),
        in_specs=[pl.BlockSpec((4, _PREP_ROWS, 128), lambda i: (0, i, 0))],
        out_specs=[
            pl.BlockSpec((16, _PREP_ROWS, 128), lambda i: (0, i, 0)),
            pl.BlockSpec((16, _PREP_ROWS, 128), lambda i: (0, i, 0)),
        ],
        out_shape=[
            jax.ShapeDtypeStruct((16, nrows, 128), jnp.int32),
            jax.ShapeDtypeStruct((16, nrows, 128), jnp.float32),
        ],
    )(xt)
    return idx3.reshape(16, _N), w3.reshape(16, _N)


def _sc_body(tab_hbm, idx_hbm, w_hbm, out_hbm, idx_v, w_v, rows_v, out_v,
             semiw0, semiw1, semg0, semg1, semo0, semo1):
    wid = lax.axis_index("s") * _NC + lax.axis_index("c")
    base = wid * _NW_Q
    semiw = (semiw0, semiw1)
    semg = (semg0, semg1)
    semo = (semo0, semo1)

    def fire_iw(j, p):
        qb = base + j * _Q
        pltpu.async_copy(idx_hbm.at[:, pl.ds(qb, _Q)], idx_v.at[p], semiw[p])
        pltpu.async_copy(w_hbm.at[:, pl.ds(qb, _Q)], w_v.at[p], semiw[p])

    def wait_iw(p):
        pltpu.make_async_copy(
            idx_hbm.at[:, pl.ds(0, _Q)], idx_v.at[p], semiw[p]).wait()
        pltpu.make_async_copy(
            w_hbm.at[:, pl.ds(0, _Q)], w_v.at[p], semiw[p]).wait()

    def fire_g(p):
        for k in range(16):
            pltpu.async_copy(tab_hbm.at[idx_v.at[p, k]], rows_v.at[p, k],
                             semg[p])

    def wait_g(p):
        for k in range(16):
            pltpu.make_async_copy(
                tab_hbm.at[pl.ds(0, _Q)], rows_v.at[p, k], semg[p]).wait()

    def fire_out(j, p):
        qb = base + j * _Q
        pltpu.async_copy(out_v.at[p], out_hbm.at[pl.ds(qb, _Q)], semo[p])

    def wait_out(p):
        pltpu.make_async_copy(
            out_hbm.at[pl.ds(0, _Q)], out_v.at[p], semo[p]).wait()

    def compute(p):
        def grp_step(g, c2):
            qb = g * 16
            wvecs = [w_v[p, k, pl.ds(qb, 16)] for k in range(16)]
            for j in range(16):
                q = qb + j
                acc_lo = jnp.zeros((16,), jnp.float32)
                acc_hi = jnp.zeros((16,), jnp.float32)
                for k in range(16):
                    wk = wvecs[k][j]
                    acc_lo = acc_lo + wk * rows_v[p, k, q, 0:16]
                    acc_hi = acc_hi + wk * rows_v[p, k, q, 16:32]
                out_v[p, q, 0:16] = acc_lo
                out_v[p, q, 16:32] = acc_hi
            return c2

        lax.fori_loop(0, _Q // 16, grp_step, 0, unroll=2)

    pltpu.sync_copy(idx_hbm.at[:, pl.ds(base, _Q)], idx_v.at[0])
    pltpu.sync_copy(w_hbm.at[:, pl.ds(base, _Q)], w_v.at[0])
    fire_g(0)
    fire_iw(1, 1)

    def pair(b2, carry):
        for ph in (0, 1):
            b = b2 * 2 + ph
            p, np_ = ph, 1 - ph
            wait_g(p)

            @pl.when(b + 1 < _NB)
            def _():
                wait_iw(np_)
                fire_g(np_)

            @pl.when(b >= 2)
            def _():
                wait_out(p)

            compute(p)
            fire_out(b, p)

            @pl.when(b + 2 < _NB)
            def _():
                fire_iw(b + 2, p)
        return carry

    lax.fori_loop(0, _NB // 2, pair, 0, unroll=False)
    wait_out(0)
    wait_out(1)


def _sc_call(tab, idx16, w16):
    mesh = plsc.VectorSubcoreMesh(core_axis_name="c", subcore_axis_name="s")
    f = functools.partial(
        pl.kernel,
        out_type=jax.ShapeDtypeStruct((_N, _F), jnp.float32),
        mesh=mesh,
        scratch_types=[
            pltpu.VMEM((2, 16, _Q), jnp.int32),
            pltpu.VMEM((2, 16, _Q), jnp.float32),
            pltpu.VMEM((2, 16, _Q, _F), jnp.float32),
            pltpu.VMEM((2, _Q, _F), jnp.float32),
            pltpu.SemaphoreType.DMA,
            pltpu.SemaphoreType.DMA,
            pltpu.SemaphoreType.DMA,
            pltpu.SemaphoreType.DMA,
            pltpu.SemaphoreType.DMA,
            pltpu.SemaphoreType.DMA,
        ],
        compiler_params=pltpu.CompilerParams(use_tc_tiling_on_sc=False),
    )(_sc_body)
    return f(tab, idx16, w16)


_LIN_R = 8192


def _lin_body(g0, g1, g2, g3, o_ref):
    o_ref[...] = jnp.concatenate(
        [g0[...], g1[...], g2[...], g3[...]], axis=1)


def _lin_call(tab2d):
    steps = _M1 // _LIN_R
    specs = [
        pl.BlockSpec((_LIN_R, _F), lambda i, c=c: (c * steps + i, 0))
        for c in range(4)
    ]
    out = pl.pallas_call(
        _lin_body,
        grid=(steps,),
        in_specs=specs,
        out_specs=pl.BlockSpec((_LIN_R, 4 * _F), lambda i: (i, 0)),
        out_shape=jax.ShapeDtypeStruct((_M1, 4 * _F), jnp.float32),
    )(tab2d, tab2d, tab2d, tab2d)
    return out.reshape(_V, _F)


def _mlp_body(v_ref, w1_ref, b1_ref, w2_ref, b2_ref, o_ref):
    h = jnp.dot(v_ref[...], w1_ref[...], preferred_element_type=jnp.float32)
    h = h + b1_ref[...]
    h = jnp.where(h >= 0, h, 0.01 * h)
    o = jnp.dot(h, w2_ref[...], preferred_element_type=jnp.float32)
    o = o + b2_ref[...]
    o_ref[...] = jax.nn.sigmoid(o) * 255.0


def _mlp_call(v, W1, b1, W2, b2):
    v4 = v.reshape(_N // 4, 4 * _F)
    eye4 = jnp.eye(4, dtype=jnp.float32)
    w1b = jnp.kron(eye4, W1)
    w2b = jnp.kron(eye4, W2)
    b1b = jnp.tile(b1, 4).reshape(1, -1)
    b2b = jnp.tile(b2, 4).reshape(1, -1)
    o4 = pl.pallas_call(
        _mlp_body,
        grid=(_N // _MLP_B,),
        in_specs=[
            pl.BlockSpec((_MLP_B // 4, 4 * _F), lambda i: (i, 0)),
            pl.BlockSpec((4 * _F, 16 * _F), lambda i: (0, 0)),
            pl.BlockSpec((1, 16 * _F), lambda i: (0, 0)),
            pl.BlockSpec((16 * _F, 12), lambda i: (0, 0)),
            pl.BlockSpec((1, 12), lambda i: (0, 0)),
        ],
        out_specs=pl.BlockSpec((_MLP_B // 4, 12), lambda i: (i, 0)),
        out_shape=jax.ShapeDtypeStruct((_N // 4, 12), jnp.float32),
    )(v4, w1b, b1b, w2b, b2b)
    return o4.reshape(_N, 3)


def kernel(x, grid, W1, b1, W2, b2):
    tab = _lin_call(grid.reshape(-1, _F))
    idx16, w16 = _prep_call(x)
    v = _sc_call(tab, idx16, w16)
    return _mlp_call(v, W1, b1, W2, b2)


# speedup vs baseline: 1.2710x; 1.2710x over previous
import functools
import math

import jax
import jax.numpy as jnp
from jax import lax
from jax.experimental import pallas as pl
from jax.experimental.pallas import tpu as pltpu
from jax.experimental.pallas import tpu_sc as plsc

_G0, _G1, _G2, _G3, _F = 64, 64, 24, 24, 32
_N = 262144
_V = _G0 * _G1 * _G2 * _G3
_M1 = _V // 4

_NC = 2
_NS = 16
_NW = _NC * _NS
_Q = 64
_NW_Q = _N // _NW
_NB = _NW_Q // _Q

_PREP_B = 2048
_MLP_B = 16384


def _prep_body(x_ref, idx_ref, w_ref):
    pi = math.pi
    x0 = x_ref[0]
    x1 = x_ref[1]
    x2 = x_ref[2]
    x3 = x_ref[3]
    t0 = (x0 - 0.0) / (pi - 0.0) * (_G0 - 1)
    t1 = (x1 - (-pi)) / (pi - (-pi)) * (_G1 - 1)
    t2 = (x2 - 0.5 * pi) / (0.85 * pi - 0.5 * pi) * (_G2 - 1)
    t3 = (x3 - (-0.85 * pi)) / (-0.5 * pi - (-0.85 * pi)) * (_G3 - 1)
    tlx = t0.astype(jnp.int32)
    tly = t1.astype(jnp.int32)
    tlz = t2.astype(jnp.int32)
    tlw = t3.astype(jnp.int32)
    xf = t0 % 1
    yf = t1 % 1
    zf = t2 % 1
    wf = t3 % 1
    brx = jnp.minimum(tlx + 1, _G1 - 1)
    bry = jnp.minimum(tly + 1, _G0 - 1)
    brz = jnp.minimum(tlz + 1, _G2 - 1)
    brw = jnp.minimum(tlw + 1, _G3 - 1)
    ws = []
    for k in range(16):
        bx, by, bz, bw = (k >> 3) & 1, (k >> 2) & 1, (k >> 1) & 1, k & 1
        iy = bry if by else tly
        ix = brx if bx else tlx
        iz = brz if bz else tlz
        iw = brw if bw else tlw
        flat = ((iy * _G1 + ix) * _G2 + iz) * _G3 + iw
        c = ((flat >= _M1).astype(jnp.int32)
             + (flat >= 2 * _M1).astype(jnp.int32)
             + (flat >= 3 * _M1).astype(jnp.int32))
        flat = (flat - c * _M1) * 4 + c
        ax = xf if bx else 1.0 - xf
        ay = yf if by else 1.0 - yf
        zsel, wsel = (bw, bz) if by else (bz, bw)
        az = zf if zsel else 1.0 - zf
        aw = wf if wsel else 1.0 - wf
        idx_ref[k] = flat
        w_ref[k] = ax * ay * az * aw


_PREP_ROWS = 64


def _prep_call(x):
    nrows = _N // 128
    xt = x.T.reshape(4, nrows, 128)
    idx3, w3 = pl.pallas_call(
        _prep_body,
        grid=(nrows // _PREP_ROWS,),
        in_specs=[pl.BlockSpec((4, _PREP_ROWS, 128), lambda i: (0, i, 0))],
        out_specs=[
            pl.BlockSpec((16, _PREP_ROWS, 128), lambda i: (0, i, 0)),
            pl.BlockSpec((16, _PREP_ROWS, 128), lambda i: (0, i, 0)),
        ],
        out_shape=[
            jax.ShapeDtypeStruct((16, nrows, 128), jnp.int32),
            jax.ShapeDtypeStruct((16, nrows, 128), jnp.float32),
        ],
    )(xt)
    return idx3.reshape(16, _N), w3.reshape(16, _N)


def _sc_body(tab_hbm, idx_hbm, w_hbm, out_hbm, idx_v, w_v, rows_v, out_v,
             semiw0, semiw1, semg0, semg1, semo0, semo1):
    wid = lax.axis_index("s") * _NC + lax.axis_index("c")
    base = wid * _NW_Q
    semiw = (semiw0, semiw1)
    semg = (semg0, semg1)
    semo = (semo0, semo1)

    def fire_iw(j, p):
        qb = base + j * _Q
        pltpu.async_copy(idx_hbm.at[:, pl.ds(qb, _Q)], idx_v.at[p], semiw[p])
        pltpu.async_copy(w_hbm.at[:, pl.ds(qb, _Q)], w_v.at[p], semiw[p])

    def wait_iw(p):
        pltpu.make_async_copy(
            idx_hbm.at[:, pl.ds(0, _Q)], idx_v.at[p], semiw[p]).wait()
        pltpu.make_async_copy(
            w_hbm.at[:, pl.ds(0, _Q)], w_v.at[p], semiw[p]).wait()

    def fire_g(p):
        for k in range(16):
            pltpu.async_copy(tab_hbm.at[idx_v.at[p, k]], rows_v.at[p, k],
                             semg[p])

    def wait_g(p):
        for k in range(16):
            pltpu.make_async_copy(
                tab_hbm.at[pl.ds(0, _Q)], rows_v.at[p, k], semg[p]).wait()

    def fire_out(j, p):
        qb = base + j * _Q
        pltpu.async_copy(out_v.at[p], out_hbm.at[pl.ds(qb, _Q)], semo[p])

    def wait_out(p):
        pltpu.make_async_copy(
            out_hbm.at[pl.ds(0, _Q)], out_v.at[p], semo[p]).wait()

    def compute(p):
        def grp_step(g, c2):
            qb = g * 16
            wvecs = [w_v[p, k, pl.ds(qb, 16)] for k in range(16)]
            for j in range(16):
                q = qb + j
                acc_lo = jnp.zeros((16,), jnp.float32)
                acc_hi = jnp.zeros((16,), jnp.float32)
                for k in range(16):
                    wk = wvecs[k][j]
                    acc_lo = acc_lo + wk * rows_v[p, k, q, 0:16]
                    acc_hi = acc_hi + wk * rows_v[p, k, q, 16:32]
                out_v[p, q, 0:16] = acc_lo
                out_v[p, q, 16:32] = acc_hi
            return c2

        lax.fori_loop(0, _Q // 16, grp_step, 0, unroll=False)

    pltpu.sync_copy(idx_hbm.at[:, pl.ds(base, _Q)], idx_v.at[0])
    pltpu.sync_copy(w_hbm.at[:, pl.ds(base, _Q)], w_v.at[0])
    fire_g(0)
    fire_iw(1, 1)

    def pair(b2, carry):
        for ph in (0, 1):
            b = b2 * 2 + ph
            p, np_ = ph, 1 - ph
            wait_g(p)

            @pl.when(b + 1 < _NB)
            def _():
                wait_iw(np_)
                fire_g(np_)

            @pl.when(b >= 2)
            def _():
                wait_out(p)

            compute(p)
            fire_out(b, p)

            @pl.when(b + 2 < _NB)
            def _():
                fire_iw(b + 2, p)
        return carry

    lax.fori_loop(0, _NB // 2, pair, 0, unroll=False)
    wait_out(0)
    wait_out(1)


def _sc_call(tab, idx16, w16):
    mesh = plsc.VectorSubcoreMesh(core_axis_name="c", subcore_axis_name="s")
    f = functools.partial(
        pl.kernel,
        out_type=jax.ShapeDtypeStruct((_N, _F), jnp.float32),
        mesh=mesh,
        scratch_types=[
            pltpu.VMEM((2, 16, _Q), jnp.int32),
            pltpu.VMEM((2, 16, _Q), jnp.float32),
            pltpu.VMEM((2, 16, _Q, _F), jnp.float32),
            pltpu.VMEM((2, _Q, _F), jnp.float32),
            pltpu.SemaphoreType.DMA,
            pltpu.SemaphoreType.DMA,
            pltpu.SemaphoreType.DMA,
            pltpu.SemaphoreType.DMA,
            pltpu.SemaphoreType.DMA,
            pltpu.SemaphoreType.DMA,
        ],
        compiler_params=pltpu.CompilerParams(use_tc_tiling_on_sc=False),
    )(_sc_body)
    return f(tab, idx16, w16)


_LIN_R = 8192


def _lin_body(g0, g1, g2, g3, o_ref):
    o_ref[...] = jnp.concatenate(
        [g0[...], g1[...], g2[...], g3[...]], axis=1)


def _lin_call(tab2d):
    steps = _M1 // _LIN_R
    specs = [
        pl.BlockSpec((_LIN_R, _F), lambda i, c=c: (c * steps + i, 0))
        for c in range(4)
    ]
    out = pl.pallas_call(
        _lin_body,
        grid=(steps,),
        in_specs=specs,
        out_specs=pl.BlockSpec((_LIN_R, 4 * _F), lambda i: (i, 0)),
        out_shape=jax.ShapeDtypeStruct((_M1, 4 * _F), jnp.float32),
    )(tab2d, tab2d, tab2d, tab2d)
    return out.reshape(_V, _F)


def _mlp_body(v_ref, w1_ref, b1_ref, w2_ref, b2_ref, o_ref):
    h = jnp.dot(v_ref[...], w1_ref[...], preferred_element_type=jnp.float32)
    h = h + b1_ref[...]
    h = jnp.where(h >= 0, h, 0.01 * h)
    o = jnp.dot(h, w2_ref[...], preferred_element_type=jnp.float32)
    o = o + b2_ref[...]
    o_ref[...] = jax.nn.sigmoid(o) * 255.0


def _mlp_call(v, W1, b1, W2, b2):
    v4 = v.reshape(_N // 4, 4 * _F)
    eye4 = jnp.eye(4, dtype=jnp.float32)
    w1b = jnp.kron(eye4, W1)
    w2b = jnp.kron(eye4, W2)
    b1b = jnp.tile(b1, 4).reshape(1, -1)
    b2b = jnp.tile(b2, 4).reshape(1, -1)
    o4 = pl.pallas_call(
        _mlp_body,
        grid=(_N // _MLP_B,),
        in_specs=[
            pl.BlockSpec((_MLP_B // 4, 4 * _F), lambda i: (i, 0)),
            pl.BlockSpec((4 * _F, 16 * _F), lambda i: (0, 0)),
            pl.BlockSpec((1, 16 * _F), lambda i: (0, 0)),
            pl.BlockSpec((16 * _F, 12), lambda i: (0, 0)),
            pl.BlockSpec((1, 12), lambda i: (0, 0)),
        ],
        out_specs=pl.BlockSpec((_MLP_B // 4, 12), lambda i: (i, 0)),
        out_shape=jax.ShapeDtypeStruct((_N // 4, 12), jnp.float32),
    )(v4, w1b, b1b, w2b, b2b)
    return o4.reshape(_N, 3)


def kernel(x, grid, W1, b1, W2, b2):
    tab = _lin_call(grid.reshape(-1, _F))
    idx16, w16 = _prep_call(x)
    v = _sc_call(tab, idx16, w16)
    return _mlp_call(v, W1, b1, W2, b2)
